# Initial kernel scaffold; baseline (speedup 1.0000x reference)
import functools

import jax
import jax.numpy as jnp
from jax import lax
from jax.experimental import pallas as pl
from jax.experimental.pallas import tpu as pltpu
from jax.experimental.pallas import tpu_sc as plsc

U_N = 60000
I_N = 40000
N = U_N + I_N
D = 32
L_N = 3
E = 1600000

NC = 2
NS = 16
LANES = 16

OWN = N // NC
ACC_R = 50176
ZROWS = 196
E_PAD = 1638400
PER_S = E_PAD // NS
CHUNK = 1024
SUB = 128
NSUB = CHUNK // SUB
NCHUNK = PER_S // CHUNK
ROWS128 = E_PAD // SUB


def _layer_body(emb, col2, row2, val2, out,
                acc, colv, rowv, lidxv, valv, rowsbuf, zbuf, sem):
    c = lax.axis_index("c")
    s = lax.axis_index("s")

    zero16 = jnp.zeros((LANES,), jnp.float32)

    def zrow(i, carry):
        zbuf[i, 0:16] = zero16
        zbuf[i, 16:32] = zero16
        return carry

    lax.fori_loop(0, ZROWS, zrow, 0)
    for i in range(16):
        pltpu.sync_copy(zbuf, acc.at[pl.ds(s * 3136 + i * ZROWS, ZROWS)])
    plsc.subcore_barrier()

    own_base = c * OWN

    def chunk_body(ci, carry):
        eb = s * (PER_S // SUB) + ci * NSUB
        pltpu.sync_copy(col2.at[pl.ds(eb, NSUB)], colv)
        pltpu.sync_copy(row2.at[pl.ds(eb, NSUB)], rowv)
        pltpu.sync_copy(val2.at[pl.ds(eb, NSUB)], valv)

        handles = []
        for j in range(NSUB):
            handles.append(pltpu.async_copy(
                emb.at[colv.at[j]],
                rowsbuf.at[pl.ds(j * SUB, SUB)], sem))
        for h in handles:
            h.wait()

        def grp(g, carry2):
            j = g // 8
            kk = (g % 8) * LANES
            row16 = rowv[j, pl.ds(kk, LANES)]
            local = row16 - own_base
            owned = local.astype(jnp.uint32) < jnp.uint32(OWN)
            dummy = OWN + (row16 & 127)
            lidxv[j, pl.ds(kk, LANES)] = jnp.where(owned, local, dummy)
            for t in range(LANES):
                e = g * LANES + t
                v = valv[j, kk + t]
                vv = jnp.full((LANES,), v, jnp.float32)
                rowsbuf[e, 0:16] = rowsbuf[e, 0:16] * vv
                rowsbuf[e, 16:32] = rowsbuf[e, 16:32] * vv
            return carry2

        lax.fori_loop(0, CHUNK // LANES, grp, 0)

        for j in range(NSUB):
            pltpu.sync_copy(rowsbuf.at[pl.ds(j * SUB, SUB)],
                            acc.at[lidxv.at[j]], add=True)
        return carry

    lax.fori_loop(0, NCHUNK, chunk_body, 0)
    plsc.subcore_barrier()

    wb = OWN // NS
    for i in range(5):
        seg = wb // 5
        src = pl.ds(s * wb + i * seg, seg)
        dst = pl.ds(own_base + s * wb + i * seg, seg)
        pltpu.sync_copy(acc.at[src], rowsbuf.at[pl.ds(0, seg)])
        pltpu.sync_copy(rowsbuf.at[pl.ds(0, seg)], out.at[dst])


_layer = pl.kernel(
    _layer_body,
    out_type=jax.ShapeDtypeStruct((N, D), jnp.float32),
    mesh=plsc.VectorSubcoreMesh(core_axis_name="c", subcore_axis_name="s"),
    scratch_types=[
        pltpu.VMEM_SHARED((ACC_R, D), jnp.float32),
        pltpu.VMEM((NSUB, SUB), jnp.int32),
        pltpu.VMEM((NSUB, SUB), jnp.int32),
        pltpu.VMEM((NSUB, SUB), jnp.int32),
        pltpu.VMEM((NSUB, SUB), jnp.float32),
        pltpu.VMEM((CHUNK, D), jnp.float32),
        pltpu.VMEM((ZROWS, D), jnp.float32),
        pltpu.SemaphoreType.DMA,
    ],
)


def _mean4_body(a, b, c, d, o):
    o[...] = (a[...] + b[...] + c[...] + d[...]) * jnp.float32(0.25)


def _mean4(e0, e1, e2, e3):
    blk = (12500, D)
    grid = (N // blk[0],)
    spec = pl.BlockSpec(blk, lambda i: (i, 0))
    return pl.pallas_call(
        _mean4_body,
        grid=grid,
        in_specs=[spec, spec, spec, spec],
        out_specs=spec,
        out_shape=jax.ShapeDtypeStruct((N, D), jnp.float32),
    )(e0, e1, e2, e3)


def kernel(edge_index, edge_values, user_emb, item_emb):
    emb0 = jnp.concatenate([user_emb, item_emb], axis=0)
    row = edge_index[0]
    col = edge_index[1]
    pad = E_PAD - E
    col2 = jnp.pad(col, (0, pad)).reshape(ROWS128, SUB)
    row2 = jnp.pad(row, (0, pad)).reshape(ROWS128, SUB)
    val2 = jnp.pad(edge_values, (0, pad)).reshape(ROWS128, SUB)

    embs = [emb0]
    cur = emb0
    for _ in range(L_N):
        cur = _layer(cur, col2, row2, val2)
        embs.append(cur)
    out = _mean4(*embs)
    return (emb0, out)


# speedup vs baseline: 5.6870x; 5.6870x over previous
import functools

import jax
import jax.numpy as jnp
from jax import lax
from jax.experimental import pallas as pl
from jax.experimental.pallas import tpu as pltpu
from jax.experimental.pallas import tpu_sc as plsc

U_N = 60000
I_N = 40000
N = U_N + I_N
D = 32
L_N = 3
E = 1600000

NC = 2
NS = 16
LANES = 16

OWN = N // NC
ACC_R = 50176
E_PAD = 1638400
PER_S = E_PAD // NS
CHUNK = 512
SUB = 128
NSUB = CHUNK // SUB
NCHUNK = PER_S // CHUNK
ROWS128 = E_PAD // SUB


def _layer_body(emb, col2, row2, val2, out,
                acc, colv, rowv, lidxv, valv, rowsbuf, sem):
    c = lax.axis_index("c")
    s = lax.axis_index("s")

    zero16 = jnp.zeros((LANES,), jnp.float32)

    def zrow(i, carry):
        rowsbuf[i, 0:16] = zero16
        rowsbuf[i, 16:32] = zero16
        return carry

    lax.fori_loop(0, 392, zrow, 0)
    for i in range(8):
        pltpu.sync_copy(rowsbuf.at[pl.ds(0, 392)],
                        acc.at[pl.ds(s * 3136 + i * 392, 392)])
    plsc.subcore_barrier()

    own_base = c * OWN

    def chunk_body(ci, carry):
        eb = s * (PER_S // SUB) + ci * NSUB
        pltpu.sync_copy(col2.at[pl.ds(eb, NSUB)], colv)
        pltpu.sync_copy(row2.at[pl.ds(eb, NSUB)], rowv)
        pltpu.sync_copy(val2.at[pl.ds(eb, NSUB)], valv)

        handles = []
        for j in range(NSUB):
            handles.append(pltpu.async_copy(
                emb.at[colv.at[j]],
                rowsbuf.at[pl.ds(j * SUB, SUB)], sem))
        for h in handles:
            h.wait()

        def grp(g, carry2):
            j = g // 8
            kk = (g % 8) * LANES
            row16 = rowv[j, pl.ds(kk, LANES)]
            local = row16 - own_base
            owned = local.astype(jnp.uint32) < jnp.uint32(OWN)
            dummy = OWN + (row16 & 127)
            lidxv[j, pl.ds(kk, LANES)] = jnp.where(owned, local, dummy)
            val16 = valv[j, pl.ds(kk, LANES)]
            for t in range(LANES):
                e = g * LANES + t
                vv = jnp.full((LANES,), val16[t], jnp.float32)
                rowsbuf[e, 0:16] = rowsbuf[e, 0:16] * vv
                rowsbuf[e, 16:32] = rowsbuf[e, 16:32] * vv
            return carry2

        lax.fori_loop(0, CHUNK // LANES, grp, 0)

        for j in range(NSUB):
            pltpu.sync_copy(rowsbuf.at[pl.ds(j * SUB, SUB)],
                            acc.at[lidxv.at[j]], add=True)
        return carry

    lax.fori_loop(0, NCHUNK, chunk_body, 0)
    plsc.subcore_barrier()

    SEG = 400
    for k in range(8):
        seg_id = s + k * NS

        @pl.when(seg_id < OWN // SEG)
        def _():
            src = pl.ds(seg_id * SEG, SEG)
            dst = pl.ds(own_base + seg_id * SEG, SEG)
            pltpu.sync_copy(acc.at[src], rowsbuf.at[pl.ds(0, SEG)])
            pltpu.sync_copy(rowsbuf.at[pl.ds(0, SEG)], out.at[dst])


_layer = pl.kernel(
    _layer_body,
    out_type=jax.ShapeDtypeStruct((N, D), jnp.float32),
    mesh=plsc.VectorSubcoreMesh(core_axis_name="c", subcore_axis_name="s"),
    compiler_params=pltpu.CompilerParams(use_tc_tiling_on_sc=False),
    scratch_types=[
        pltpu.VMEM_SHARED((ACC_R, D), jnp.float32),
        pltpu.VMEM((NSUB, SUB), jnp.int32),
        pltpu.VMEM((NSUB, SUB), jnp.int32),
        pltpu.VMEM((NSUB, SUB), jnp.int32),
        pltpu.VMEM((NSUB, SUB), jnp.float32),
        pltpu.VMEM((CHUNK, D), jnp.float32),
        pltpu.SemaphoreType.DMA,
    ],
)


def _mean4_body(a, b, c, d, o):
    o[...] = (a[...] + b[...] + c[...] + d[...]) * jnp.float32(0.25)


def _mean4(e0, e1, e2, e3):
    blk = (5000, D)
    grid = (N // blk[0],)
    spec = pl.BlockSpec(blk, lambda i: (i, 0))
    return pl.pallas_call(
        _mean4_body,
        grid=grid,
        in_specs=[spec, spec, spec, spec],
        out_specs=spec,
        out_shape=jax.ShapeDtypeStruct((N, D), jnp.float32),
    )(e0, e1, e2, e3)


def kernel(edge_index, edge_values, user_emb, item_emb):
    emb0 = jnp.concatenate([user_emb, item_emb], axis=0)
    row = edge_index[0]
    col = edge_index[1]
    pad = E_PAD - E
    col2 = jnp.pad(col, (0, pad)).reshape(ROWS128, SUB)
    row2 = jnp.pad(row, (0, pad)).reshape(ROWS128, SUB)
    val2 = jnp.pad(edge_values, (0, pad)).reshape(ROWS128, SUB)

    embs = [emb0]
    cur = emb0
    for _ in range(L_N):
        cur = _layer(cur, col2, row2, val2)
        embs.append(cur)
    out = _mean4(*embs)
    return (emb0, out)
